# baseline (speedup 1.0000x reference)
import functools

import jax
import jax.numpy as jnp
from jax import lax
from jax.experimental import pallas as pl
from jax.experimental.pallas import tpu as pltpu
from jax.experimental.pallas import tpu_sc as plsc

B = 32
S = 2048
H = 64
V = 1000000
L = 16

_info = plsc.get_sparse_core_info()
NC, NS = _info.num_cores, _info.num_subcores
NW = NC * NS
S_SLAB = 128
N_SLABS = S // S_SLAB
B_PER_W = B // 2
NBUF = 3


def _body(ids_hbm, tok_hbm, pos_hbm, out_hbm, idx_v, idx2_v, par_v, pos_v,
          rows, stages, sems_g, sems_w, sem_p):
    wid = lax.axis_index("s") * NC + lax.axis_index("c")
    slab = wid // 2
    half = wid % 2
    s0 = slab * S_SLAB

    cp = pltpu.async_copy(pos_hbm.at[:, pl.ds(s0, S_SLAB)], pos_v, sem_p)
    ci = pltpu.async_copy(
        ids_hbm.at[pl.ds(half * B_PER_W, B_PER_W), pl.ds(s0, S_SLAB)],
        idx_v, sem_p)
    cp.wait()
    ci.wait()

    for r in range(B_PER_W):
        for k in range(S_SLAB // L):
            v = idx_v[r, pl.ds(k * L, L)]
            idx2_v[r, pl.ds(k * L, L)] = jnp.right_shift(v, 1)
            par_v[r, pl.ds(k * L, L)] = jnp.left_shift(
                jnp.bitwise_and(v, 1), 6)

    def fire_gather(b):
        p = b % NBUF
        return pltpu.async_copy(tok_hbm.at[idx2_v.at[b]], rows[p], sems_g[p])

    iota = lax.iota(jnp.int32, L)
    gathers = {b: fire_gather(b) for b in range(NBUF)}
    writes = {}
    for b in range(B_PER_W):
        p = b % NBUF
        q = b % 2
        if b >= 2:
            writes.pop(b - 2).wait()
        gathers.pop(b).wait()

        rows_p, stage_q = rows[p], stages[q]

        def select_add(sg, c, rows_p=rows_p, stage_q=stage_q, b=b):
            par64 = par_v[b, pl.ds(sg * L, L)]
            riota = sg * L + iota
            for j in range(H):
                g = plsc.load_gather(rows_p, [riota, par64 + j])
                stage_q[j, pl.ds(sg * L, L)] = g + pos_v[j, pl.ds(sg * L, L)]
            return c

        lax.fori_loop(0, S_SLAB // L, select_add, 0)

        bb = half * B_PER_W + b
        writes[b] = pltpu.async_copy(
            stage_q, out_hbm.at[bb, :, pl.ds(s0, S_SLAB)], sems_w[q])
        nxt = b + NBUF
        if nxt < B_PER_W:
            gathers[nxt] = fire_gather(nxt)
    for b in sorted(writes):
        writes[b].wait()


def kernel(input_ids, token_table, pos_table):
    mesh = plsc.VectorSubcoreMesh(core_axis_name="c", subcore_axis_name="s")

    def body(ids_hbm, tok_hbm, pos_hbm, out_hbm, idx_v, idx2_v, par_v, pos_v,
             r0, r1, r2, st0, st1, g0, g1, g2, w0, w1, sem_p):
        _body(ids_hbm, tok_hbm, pos_hbm, out_hbm, idx_v, idx2_v, par_v, pos_v,
              [r0, r1, r2], [st0, st1], [g0, g1, g2], [w0, w1], sem_p)

    k = functools.partial(
        pl.kernel,
        mesh=mesh,
        out_type=jax.ShapeDtypeStruct((B, H, S), jnp.float32),
        scratch_types=(
            [pltpu.VMEM((B_PER_W, S_SLAB), jnp.int32) for _ in range(3)]
            + [pltpu.VMEM((H, S_SLAB), jnp.float32)]
            + [pltpu.VMEM((S_SLAB, 2 * H), jnp.float32) for _ in range(NBUF)]
            + [pltpu.VMEM((H, S_SLAB), jnp.float32) for _ in range(2)]
            + [pltpu.SemaphoreType.DMA for _ in range(NBUF + 2 + 1)]
        ),
        compiler_params=pltpu.CompilerParams(needs_layout_passes=False),
    )(body)
    tok2 = token_table.reshape(V // 2, 2 * H)
    out_t = k(input_ids, tok2, pos_table.T)
    return jnp.transpose(out_t, (0, 2, 1))


# speedup vs baseline: 1.0971x; 1.0971x over previous
import functools

import jax
import jax.numpy as jnp
from jax import lax
from jax.experimental import pallas as pl
from jax.experimental.pallas import tpu as pltpu
from jax.experimental.pallas import tpu_sc as plsc

B = 32
S = 2048
H = 64
V = 1000000
L = 16

_info = plsc.get_sparse_core_info()
NC, NS = _info.num_cores, _info.num_subcores
NW = NC * NS
S_SLAB = 128
N_SLABS = S // S_SLAB
B_PER_W = B // 2
NBUF = 3


def _body(ids_hbm, tok_hbm, pos_hbm, out_hbm, idx_v, pos_v, rows, stages,
          sems_g, sems_w, sem_p):
    wid = lax.axis_index("s") * NC + lax.axis_index("c")
    slab = wid // 2
    half = wid % 2
    s0 = slab * S_SLAB

    cp = pltpu.async_copy(pos_hbm.at[:, pl.ds(s0, S_SLAB)], pos_v, sem_p)
    ci = pltpu.async_copy(
        ids_hbm.at[pl.ds(half * B_PER_W, B_PER_W), pl.ds(s0, S_SLAB)],
        idx_v, sem_p)
    cp.wait()
    ci.wait()

    def fire_gather(b):
        p = b % NBUF
        return pltpu.async_copy(tok_hbm.at[idx_v.at[b]], rows[p], sems_g[p])

    iota = lax.iota(jnp.int32, L)
    gathers = {b: fire_gather(b) for b in range(NBUF)}
    writes = {}
    for b in range(B_PER_W):
        p = b % NBUF
        q = b % 2
        if b >= 2:
            writes.pop(b - 2).wait()
        gathers.pop(b).wait()

        rows_p, stage_q = rows[p], stages[q]

        def select_add(sg, c, rows_p=rows_p, stage_q=stage_q):
            riota = sg * L + iota
            for j in range(H):
                g = plsc.load_gather(rows_p, [riota, jnp.full((L,), j,
                                                             jnp.int32)])
                stage_q[j, pl.ds(sg * L, L)] = g + pos_v[j, pl.ds(sg * L, L)]
            return c

        lax.fori_loop(0, S_SLAB // L, select_add, 0)

        bb = half * B_PER_W + b
        writes[b] = pltpu.async_copy(
            stage_q, out_hbm.at[bb, :, pl.ds(s0, S_SLAB)], sems_w[q])
        nxt = b + NBUF
        if nxt < B_PER_W:
            gathers[nxt] = fire_gather(nxt)
    for b in sorted(writes):
        writes[b].wait()


def kernel(input_ids, token_table, pos_table):
    mesh = plsc.VectorSubcoreMesh(core_axis_name="c", subcore_axis_name="s")

    def body(ids_hbm, tok_hbm, pos_hbm, out_hbm, idx_v, pos_v,
             r0, r1, r2, st0, st1, g0, g1, g2, w0, w1, sem_p):
        _body(ids_hbm, tok_hbm, pos_hbm, out_hbm, idx_v, pos_v,
              [r0, r1, r2], [st0, st1], [g0, g1, g2], [w0, w1], sem_p)

    k = functools.partial(
        pl.kernel,
        mesh=mesh,
        out_type=jax.ShapeDtypeStruct((B, H, S), jnp.float32),
        scratch_types=(
            [pltpu.VMEM((B_PER_W, S_SLAB), jnp.int32),
             pltpu.VMEM((H, S_SLAB), jnp.float32)]
            + [pltpu.VMEM((S_SLAB, 2 * H), jnp.float32) for _ in range(NBUF)]
            + [pltpu.VMEM((H, S_SLAB), jnp.float32) for _ in range(2)]
            + [pltpu.SemaphoreType.DMA for _ in range(NBUF + 2 + 1)]
        ),
        compiler_params=pltpu.CompilerParams(needs_layout_passes=False),
    )(body)
    tok_pad = jnp.pad(token_table, ((0, 0), (0, H)))
    out_t = k(input_ids, tok_pad, pos_table.T)
    return jnp.transpose(out_t, (0, 2, 1))
